# Initial kernel scaffold; baseline (speedup 1.0000x reference)
import jax
import jax.numpy as jnp
from jax.experimental import pallas as pl


def kernel(x, edge_index, W1, b1, W2, b2):
    raise NotImplementedError("write your pallas kernel here")


# speedup vs baseline: 8.2343x; 8.2343x over previous
import functools

import jax
import jax.numpy as jnp
from jax import lax
from jax.experimental import pallas as pl
from jax.experimental.pallas import tpu as pltpu
from jax.experimental.pallas import tpu_sc as plsc

N = 10000
E = 320000
NC, NS, L = 2, 16, 16
NW = NC * NS
EPT = E // NW
K = 80
STEPS = EPT // K
NROWS_A = 640


def _make_sc_pass(count_mode, nrows, depth):
    rpt = nrows // NS

    def body(table_h, src_h, dst_h, out_h, sidx, didx, msg, zbuf, acc):
        cid = lax.axis_index("c")
        sid = lax.axis_index("s")
        wid = cid * NS + sid

        zero = jnp.zeros((L,), jnp.float32)

        def zrow(r, carry):
            for c in range(depth // L):
                zbuf[r, pl.ds(c * L, L)] = zero
            return carry

        lax.fori_loop(0, rpt, zrow, 0)
        pltpu.sync_copy(zbuf, acc.at[pl.ds(sid * rpt, rpt)])

        pltpu.sync_copy(dst_h.at[wid], didx)
        if count_mode:
            def split(j, carry):
                for i in range(K // L):
                    q = didx[j, pl.ds(i * L, L)]
                    sidx[j, pl.ds(i * L, L)] = lax.bitwise_and(q, 15)
                    didx[j, pl.ds(i * L, L)] = lax.shift_right_logical(q, 4)
                return carry

            lax.fori_loop(0, STEPS, split, 0)
        else:
            pltpu.sync_copy(src_h.at[wid], sidx)

        plsc.subcore_barrier()

        def step(j, carry):
            pltpu.sync_copy(table_h.at[sidx.at[j]], msg)
            pltpu.sync_copy(msg, acc.at[didx.at[j]], add=True)
            return carry

        lax.fori_loop(0, STEPS, step, 0)

        plsc.subcore_barrier()
        pltpu.sync_copy(acc.at[pl.ds(sid * rpt, rpt)], out_h.at[wid])

    mesh = plsc.VectorSubcoreMesh(
        core_axis_name="c", subcore_axis_name="s",
        num_cores=NC, num_subcores=NS)
    return pl.kernel(
        body,
        out_type=jax.ShapeDtypeStruct((NW, rpt, depth), jnp.float32),
        mesh=mesh,
        compiler_params=pltpu.CompilerParams(use_tc_tiling_on_sc=False),
        scratch_types=[
            pltpu.VMEM((STEPS, K), jnp.int32),
            pltpu.VMEM((STEPS, K), jnp.int32),
            pltpu.VMEM((K, depth), jnp.float32),
            pltpu.VMEM((rpt, depth), jnp.float32),
            pltpu.VMEM_SHARED((nrows, depth), jnp.float32),
        ],
    )


_sc_count = _make_sc_pass(True, NROWS_A, 16)
_sc_msg = _make_sc_pass(False, N, 64)

RB = 1000
G = N // RB


def _tc1_body(x_r, w_r, ca_r, cb_r, h_r, dinv_r):
    deg = ca_r[...] + cb_r[...] + 1.0
    dinv = lax.rsqrt(deg)
    h = jnp.dot(x_r[...], w_r[...], preferred_element_type=jnp.float32)
    dinv_r[...] = dinv
    h_r[...] = h * dinv


_tc1 = pl.pallas_call(
    _tc1_body,
    grid=(G,),
    in_specs=[
        pl.BlockSpec((RB, 128), lambda i: (i, 0)),
        pl.BlockSpec((128, 64), lambda i: (0, 0)),
        pl.BlockSpec((RB, 1), lambda i: (i, 0)),
        pl.BlockSpec((RB, 1), lambda i: (i, 0)),
    ],
    out_specs=[
        pl.BlockSpec((RB, 64), lambda i: (i, 0)),
        pl.BlockSpec((RB, 1), lambda i: (i, 0)),
    ],
    out_shape=[
        jax.ShapeDtypeStruct((N, 64), jnp.float32),
        jax.ShapeDtypeStruct((N, 1), jnp.float32),
    ],
)


def _tc2_body(sa_r, sb_r, hp_r, dinv_r, b1_r, w2_r, out_r):
    dinv = dinv_r[...]
    z = dinv * (sa_r[...] + sb_r[...] + hp_r[...]) + b1_r[...]
    z = jnp.maximum(z, 0.0)
    h2 = jnp.dot(z, w2_r[...], preferred_element_type=jnp.float32)
    out_r[...] = h2 * dinv


_tc2 = pl.pallas_call(
    _tc2_body,
    grid=(G,),
    in_specs=[
        pl.BlockSpec((RB, 64), lambda i: (i, 0)),
        pl.BlockSpec((RB, 64), lambda i: (i, 0)),
        pl.BlockSpec((RB, 64), lambda i: (i, 0)),
        pl.BlockSpec((RB, 1), lambda i: (i, 0)),
        pl.BlockSpec((1, 64), lambda i: (0, 0)),
        pl.BlockSpec((64, 64), lambda i: (0, 0)),
    ],
    out_specs=pl.BlockSpec((RB, 64), lambda i: (i, 0)),
    out_shape=jax.ShapeDtypeStruct((N, 64), jnp.float32),
)


def _tc3_body(sa_r, sb_r, hp_r, dinv_r, b2_r, out_r):
    z = dinv_r[...] * (sa_r[...] + sb_r[...] + hp_r[...]) + b2_r[...]
    col = lax.broadcasted_iota(jnp.int32, (RB, 64), 1)
    zm = jnp.where(col < 40, z, -1e30)
    m = jnp.max(zm, axis=1, keepdims=True)
    e = jnp.exp(zm - m)
    s = jnp.sum(e, axis=1, keepdims=True)
    out_r[...] = zm - m - jnp.log(s)


_tc3 = pl.pallas_call(
    _tc3_body,
    grid=(G,),
    in_specs=[
        pl.BlockSpec((RB, 64), lambda i: (i, 0)),
        pl.BlockSpec((RB, 64), lambda i: (i, 0)),
        pl.BlockSpec((RB, 64), lambda i: (i, 0)),
        pl.BlockSpec((RB, 1), lambda i: (i, 0)),
        pl.BlockSpec((1, 64), lambda i: (0, 0)),
    ],
    out_specs=pl.BlockSpec((RB, 64), lambda i: (i, 0)),
    out_shape=jax.ShapeDtypeStruct((N, 64), jnp.float32),
)


@jax.jit
def kernel(x, edge_index, W1, b1, W2, b2):
    ei = edge_index.astype(jnp.int32)
    src = ei[0].reshape(NW, STEPS, K)
    dst = ei[1].reshape(NW, STEPS, K)

    id16 = jnp.eye(16, dtype=jnp.float32)
    cnt = _sc_count(id16, dst, dst)
    cnt = cnt.reshape(NC, NS * (NROWS_A // NS) * 16)[:, :N]
    ca = cnt[0].reshape(N, 1)
    cb = cnt[1].reshape(N, 1)

    h1p, dinv = _tc1(x, W1, ca, cb)

    s1 = _sc_msg(h1p, src, dst).reshape(NC, N, 64)

    b1r = b1.reshape(1, 64)
    w2p = jnp.concatenate(
        [W2, jnp.zeros((64, 24), jnp.float32)], axis=1)
    b2p = jnp.concatenate([b2, jnp.zeros((24,), jnp.float32)]).reshape(1, 64)

    h2p = _tc2(s1[0], s1[1], h1p, dinv, b1r, w2p)

    s2 = _sc_msg(h2p, src, dst).reshape(NC, N, 64)

    out = _tc3(s2[0], s2[1], h2p, dinv, b2p)
    return out[:, :40]


# speedup vs baseline: 26.0321x; 3.1614x over previous
import functools

import jax
import jax.numpy as jnp
from jax import lax
from jax.experimental import pallas as pl
from jax.experimental.pallas import tpu as pltpu
from jax.experimental.pallas import tpu_sc as plsc

N = 10000
E = 320000
NC, NS, L = 2, 16, 16
NW = NC * NS
EPT = E // NW
K = 80
STEPS = EPT // K
NROWS_A = 640


_MESH = plsc.VectorSubcoreMesh(
    core_axis_name="c", subcore_axis_name="s",
    num_cores=NC, num_subcores=NS)
_SC_PARAMS = pltpu.CompilerParams(use_tc_tiling_on_sc=False)


def _zero_rows(buf, nrows_buf, ncols):
    zero = jnp.zeros((L,), jnp.float32)

    def zrow(r, carry):
        for c in range(ncols // L):
            buf[r, pl.ds(c * L, L)] = zero
        return carry

    lax.fori_loop(0, nrows_buf, zrow, 0)


ACC_A = NROWS_A * 16
EPT_A = ACC_A // NS
CNT_WIN = 8


def _count_body(dst_h, out_h, didx, ones_v, zbuf, acc, sem):
    cid = lax.axis_index("c")
    sid = lax.axis_index("s")
    wid = cid * NS + sid

    zero = jnp.zeros((L,), jnp.float32)
    one = jnp.ones((L,), jnp.float32)

    def fill(r, carry):
        zbuf[pl.ds(r * L, L)] = zero
        return carry

    lax.fori_loop(0, EPT_A // L, fill, 0)
    for i in range(K // L):
        ones_v[pl.ds(i * L, L)] = one
    pltpu.sync_copy(zbuf, acc.at[pl.ds(sid * EPT_A, EPT_A)])

    pltpu.sync_copy(dst_h.at[wid], didx)
    plsc.subcore_barrier()

    def s_start(j):
        pltpu.async_copy(ones_v, acc.at[didx.at[j]], sem, add=True)

    def s_wait(j):
        pltpu.make_async_copy(ones_v, acc.at[didx.at[j]], sem).wait()

    def step(j, carry):
        s_start(j)

        @pl.when(j >= CNT_WIN)
        def _():
            s_wait(j - CNT_WIN)

        return carry

    lax.fori_loop(0, STEPS, step, 0)
    for j in range(STEPS - CNT_WIN, STEPS):
        s_wait(j)

    plsc.subcore_barrier()
    pltpu.sync_copy(acc.at[pl.ds(sid * EPT_A, EPT_A)], out_h.at[wid])


_sc_count = pl.kernel(
    _count_body,
    out_type=jax.ShapeDtypeStruct((NW, EPT_A), jnp.float32),
    mesh=_MESH,
    compiler_params=_SC_PARAMS,
    scratch_types=[
        pltpu.VMEM((STEPS, K), jnp.int32),
        pltpu.VMEM((K,), jnp.float32),
        pltpu.VMEM((EPT_A,), jnp.float32),
        pltpu.VMEM_SHARED((ACC_A,), jnp.float32),
        pltpu.SemaphoreType.DMA,
    ],
)

D = 64


def _msg_body(table_h, src_h, dst_h, out_h, sidx, didx, msg, zbuf, acc,
              gsem, ssem):
    cid = lax.axis_index("c")
    sid = lax.axis_index("s")
    wid = cid * NS + sid
    rpt = N // NS

    _zero_rows(zbuf, rpt, D)
    pltpu.sync_copy(zbuf, acc.at[pl.ds(sid * rpt, rpt)])

    pltpu.sync_copy(src_h.at[wid], sidx)
    pltpu.sync_copy(dst_h.at[wid], didx)
    plsc.subcore_barrier()

    def g_start(j, b):
        pltpu.async_copy(table_h.at[sidx.at[j]], msg.at[b], gsem.at[b])

    def g_wait(j, b):
        pltpu.make_async_copy(
            table_h.at[sidx.at[j]], msg.at[b], gsem.at[b]).wait()

    def s_start(j, b):
        pltpu.async_copy(msg.at[b], acc.at[didx.at[j]], ssem.at[b], add=True)

    def s_wait(j, b):
        pltpu.make_async_copy(
            msg.at[b], acc.at[didx.at[j]], ssem.at[b]).wait()

    g_start(0, 0)

    def pair(i, carry):
        j0 = 2 * i
        j1 = j0 + 1
        g_wait(j0, 0)
        s_start(j0, 0)

        @pl.when(i >= 1)
        def _():
            s_wait(j0 - 1, 1)

        g_start(j1, 1)
        g_wait(j1, 1)
        s_start(j1, 1)
        s_wait(j0, 0)

        @pl.when(j1 + 1 < STEPS)
        def _():
            g_start(j1 + 1, 0)

        return carry

    lax.fori_loop(0, STEPS // 2, pair, 0)
    jt = STEPS - 1
    g_wait(jt, 0)
    s_start(jt, 0)
    s_wait(jt - 1, 1)
    s_wait(jt, 0)

    plsc.subcore_barrier()
    pltpu.sync_copy(acc.at[pl.ds(sid * rpt, rpt)], out_h.at[wid])


_sc_msg = pl.kernel(
    _msg_body,
    out_type=jax.ShapeDtypeStruct((NW, N // NS, D), jnp.float32),
    mesh=_MESH,
    compiler_params=_SC_PARAMS,
    scratch_types=[
        pltpu.VMEM((STEPS, K), jnp.int32),
        pltpu.VMEM((STEPS, K), jnp.int32),
        pltpu.VMEM((2, K, D), jnp.float32),
        pltpu.VMEM((N // NS, D), jnp.float32),
        pltpu.VMEM_SHARED((N, D), jnp.float32),
        pltpu.SemaphoreType.DMA((2,)),
        pltpu.SemaphoreType.DMA((2,)),
    ],
)

RB = 1000
G = N // RB


def _tc1_body(x_r, w_r, ca_r, cb_r, h_r, dinv_r):
    deg = ca_r[...] + cb_r[...] + 1.0
    dinv = lax.rsqrt(deg)
    h = jnp.dot(x_r[...], w_r[...], preferred_element_type=jnp.float32)
    dinv_r[...] = dinv
    h_r[...] = h * dinv


_tc1 = pl.pallas_call(
    _tc1_body,
    grid=(G,),
    in_specs=[
        pl.BlockSpec((RB, 128), lambda i: (i, 0)),
        pl.BlockSpec((128, 64), lambda i: (0, 0)),
        pl.BlockSpec((RB, 1), lambda i: (i, 0)),
        pl.BlockSpec((RB, 1), lambda i: (i, 0)),
    ],
    out_specs=[
        pl.BlockSpec((RB, 64), lambda i: (i, 0)),
        pl.BlockSpec((RB, 1), lambda i: (i, 0)),
    ],
    out_shape=[
        jax.ShapeDtypeStruct((N, 64), jnp.float32),
        jax.ShapeDtypeStruct((N, 1), jnp.float32),
    ],
)


def _tc2_body(sa_r, sb_r, hp_r, dinv_r, b1_r, w2_r, out_r):
    dinv = dinv_r[...]
    z = dinv * (sa_r[...] + sb_r[...] + hp_r[...]) + b1_r[...]
    z = jnp.maximum(z, 0.0)
    h2 = jnp.dot(z, w2_r[...], preferred_element_type=jnp.float32)
    out_r[...] = h2 * dinv


_tc2 = pl.pallas_call(
    _tc2_body,
    grid=(G,),
    in_specs=[
        pl.BlockSpec((RB, 64), lambda i: (i, 0)),
        pl.BlockSpec((RB, 64), lambda i: (i, 0)),
        pl.BlockSpec((RB, 64), lambda i: (i, 0)),
        pl.BlockSpec((RB, 1), lambda i: (i, 0)),
        pl.BlockSpec((1, 64), lambda i: (0, 0)),
        pl.BlockSpec((64, 64), lambda i: (0, 0)),
    ],
    out_specs=pl.BlockSpec((RB, 64), lambda i: (i, 0)),
    out_shape=jax.ShapeDtypeStruct((N, 64), jnp.float32),
)


def _tc3_body(sa_r, sb_r, hp_r, dinv_r, b2_r, out_r):
    z = dinv_r[...] * (sa_r[...] + sb_r[...] + hp_r[...]) + b2_r[...]
    col = lax.broadcasted_iota(jnp.int32, (RB, 64), 1)
    zm = jnp.where(col < 40, z, -1e30)
    m = jnp.max(zm, axis=1, keepdims=True)
    e = jnp.exp(zm - m)
    s = jnp.sum(e, axis=1, keepdims=True)
    out_r[...] = zm - m - jnp.log(s)


_tc3 = pl.pallas_call(
    _tc3_body,
    grid=(G,),
    in_specs=[
        pl.BlockSpec((RB, 64), lambda i: (i, 0)),
        pl.BlockSpec((RB, 64), lambda i: (i, 0)),
        pl.BlockSpec((RB, 64), lambda i: (i, 0)),
        pl.BlockSpec((RB, 1), lambda i: (i, 0)),
        pl.BlockSpec((1, 64), lambda i: (0, 0)),
    ],
    out_specs=pl.BlockSpec((RB, 64), lambda i: (i, 0)),
    out_shape=jax.ShapeDtypeStruct((N, 64), jnp.float32),
)


@jax.jit
def kernel(x, edge_index, W1, b1, W2, b2):
    ei = edge_index.astype(jnp.int32)
    src = ei[0].reshape(NW, STEPS, K)
    dst = ei[1].reshape(NW, STEPS, K)

    cnt = _sc_count(dst)
    cnt = cnt.reshape(NC, NS * EPT_A)[:, :N]
    ca = cnt[0].reshape(N, 1)
    cb = cnt[1].reshape(N, 1)

    h1p, dinv = _tc1(x, W1, ca, cb)

    s1 = _sc_msg(h1p, src, dst).reshape(NC, N, 64)

    b1r = b1.reshape(1, 64)
    w2p = jnp.concatenate(
        [W2, jnp.zeros((64, 24), jnp.float32)], axis=1)
    b2p = jnp.concatenate([b2, jnp.zeros((24,), jnp.float32)]).reshape(1, 64)

    h2p = _tc2(s1[0], s1[1], h1p, dinv, b1r, w2p)

    s2 = _sc_msg(h2p, src, dst).reshape(NC, N, 64)

    out = _tc3(s2[0], s2[1], h2p, dinv, b2p)
    return out[:, :40]


# speedup vs baseline: 38.6366x; 1.4842x over previous
import functools

import jax
import jax.numpy as jnp
from jax import lax
from jax.experimental import pallas as pl
from jax.experimental.pallas import tpu as pltpu
from jax.experimental.pallas import tpu_sc as plsc

N = 10000
E = 320000
NC, NS, L = 2, 16, 16
NW = NC * NS
EPT = E // NW
K = 80
STEPS = EPT // K
NROWS_A = 640


_MESH = plsc.VectorSubcoreMesh(
    core_axis_name="c", subcore_axis_name="s",
    num_cores=NC, num_subcores=NS)
_SC_PARAMS = pltpu.CompilerParams(use_tc_tiling_on_sc=False)


def _zero_rows(buf, nrows_buf, ncols):
    zero = jnp.zeros((L,), jnp.float32)

    def zrow(r, carry):
        for c in range(ncols // L):
            buf[r, pl.ds(c * L, L)] = zero
        return carry

    lax.fori_loop(0, nrows_buf, zrow, 0)


ACC_A = NROWS_A * 16
EPT_A = ACC_A // NS
CNT_WIN = 8


def _count_body(dst_h, out_h, didx, ones_v, zbuf, acc, sem):
    cid = lax.axis_index("c")
    sid = lax.axis_index("s")
    wid = cid * NS + sid

    zero = jnp.zeros((L,), jnp.float32)
    one = jnp.ones((L,), jnp.float32)

    def fill(r, carry):
        zbuf[pl.ds(r * L, L)] = zero
        return carry

    lax.fori_loop(0, EPT_A // L, fill, 0)
    for i in range(K // L):
        ones_v[pl.ds(i * L, L)] = one
    pltpu.sync_copy(zbuf, acc.at[pl.ds(sid * EPT_A, EPT_A)])

    pltpu.sync_copy(dst_h.at[wid], didx)
    plsc.subcore_barrier()

    def s_start(j):
        pltpu.async_copy(ones_v, acc.at[didx.at[j]], sem, add=True)

    def s_wait(j):
        pltpu.make_async_copy(ones_v, acc.at[didx.at[j]], sem).wait()

    def step(j, carry):
        s_start(j)

        @pl.when(j >= CNT_WIN)
        def _():
            s_wait(j - CNT_WIN)

        return carry

    lax.fori_loop(0, STEPS, step, 0)
    for j in range(STEPS - CNT_WIN, STEPS):
        s_wait(j)

    plsc.subcore_barrier()
    pltpu.sync_copy(acc.at[pl.ds(sid * EPT_A, EPT_A)], out_h.at[wid])


_sc_count = pl.kernel(
    _count_body,
    out_type=jax.ShapeDtypeStruct((NW, EPT_A), jnp.float32),
    mesh=_MESH,
    compiler_params=_SC_PARAMS,
    scratch_types=[
        pltpu.VMEM((STEPS, K), jnp.int32),
        pltpu.VMEM((K,), jnp.float32),
        pltpu.VMEM((EPT_A,), jnp.float32),
        pltpu.VMEM_SHARED((ACC_A,), jnp.float32),
        pltpu.SemaphoreType.DMA,
    ],
)

D = 64
NBUF = 10
LAG = 5
ZR = 125


def _msg_body(table_h, src_h, dst_h, out_h, sidx, didx, msg, zbuf, acc,
              gsem, ssem):
    cid = lax.axis_index("c")
    sid = lax.axis_index("s")
    wid = cid * NS + sid
    rpt = N // NS

    _zero_rows(zbuf, ZR, D)
    for t in range(rpt // ZR):
        pltpu.sync_copy(zbuf, acc.at[pl.ds(sid * rpt + t * ZR, ZR)])

    pltpu.sync_copy(src_h.at[wid], sidx)
    pltpu.sync_copy(dst_h.at[wid], didx)
    plsc.subcore_barrier()

    def g_start(j, b):
        pltpu.async_copy(table_h.at[sidx.at[j]], msg.at[b], gsem.at[b])

    def g_wait(j, b):
        pltpu.make_async_copy(
            table_h.at[sidx.at[j]], msg.at[b], gsem.at[b]).wait()

    def s_start(j, b):
        pltpu.async_copy(msg.at[b], acc.at[didx.at[j]], ssem.at[b], add=True)

    def s_wait(j, b):
        pltpu.make_async_copy(
            msg.at[b], acc.at[didx.at[j]], ssem.at[b]).wait()

    for j in range(LAG):
        g_start(j, j % NBUF)
    for j in range(NBUF):
        b = j % NBUF
        bn = (b + LAG) % NBUF
        if j >= LAG:
            s_wait(j - LAG, bn)
        g_start(j + LAG, bn)
        g_wait(j, b)
        s_start(j, b)

    def group(i, carry):
        j0 = i * NBUF
        for b in range(NBUF):
            j = j0 + b
            bn = (b + LAG) % NBUF
            s_wait(j - LAG, bn)
            g_start(j + LAG, bn)
            g_wait(j, b)
            s_start(j, b)
        return carry

    lax.fori_loop(1, (STEPS - LAG) // NBUF, group, 0)
    for j in range(((STEPS - LAG) // NBUF) * NBUF, STEPS):
        b = j % NBUF
        s_wait(j - LAG, (b + LAG) % NBUF)
        if j + LAG < STEPS:
            g_start(j + LAG, (b + LAG) % NBUF)
        g_wait(j, b)
        s_start(j, b)
    for j in range(STEPS - LAG, STEPS):
        s_wait(j, j % NBUF)

    plsc.subcore_barrier()
    pltpu.sync_copy(acc.at[pl.ds(sid * rpt, rpt)], out_h.at[wid])


_sc_msg = pl.kernel(
    _msg_body,
    out_type=jax.ShapeDtypeStruct((NW, N // NS, D), jnp.float32),
    mesh=_MESH,
    compiler_params=_SC_PARAMS,
    scratch_types=[
        pltpu.VMEM((STEPS, K), jnp.int32),
        pltpu.VMEM((STEPS, K), jnp.int32),
        pltpu.VMEM((NBUF, K, D), jnp.float32),
        pltpu.VMEM((ZR, D), jnp.float32),
        pltpu.VMEM_SHARED((N, D), jnp.float32),
        pltpu.SemaphoreType.DMA((NBUF,)),
        pltpu.SemaphoreType.DMA((NBUF,)),
    ],
)

RB = 1000
G = N // RB


def _tc1_body(x_r, w_r, ca_r, cb_r, h_r, dinv_r):
    deg = ca_r[...] + cb_r[...] + 1.0
    dinv = lax.rsqrt(deg)
    h = jnp.dot(x_r[...], w_r[...], preferred_element_type=jnp.float32)
    dinv_r[...] = dinv
    h_r[...] = h * dinv


_tc1 = pl.pallas_call(
    _tc1_body,
    grid=(G,),
    in_specs=[
        pl.BlockSpec((RB, 128), lambda i: (i, 0)),
        pl.BlockSpec((128, 64), lambda i: (0, 0)),
        pl.BlockSpec((RB, 1), lambda i: (i, 0)),
        pl.BlockSpec((RB, 1), lambda i: (i, 0)),
    ],
    out_specs=[
        pl.BlockSpec((RB, 64), lambda i: (i, 0)),
        pl.BlockSpec((RB, 1), lambda i: (i, 0)),
    ],
    out_shape=[
        jax.ShapeDtypeStruct((N, 64), jnp.float32),
        jax.ShapeDtypeStruct((N, 1), jnp.float32),
    ],
)


def _tc2_body(sa_r, sb_r, hp_r, dinv_r, b1_r, w2_r, out_r):
    dinv = dinv_r[...]
    z = dinv * (sa_r[...] + sb_r[...] + hp_r[...]) + b1_r[...]
    z = jnp.maximum(z, 0.0)
    h2 = jnp.dot(z, w2_r[...], preferred_element_type=jnp.float32)
    out_r[...] = h2 * dinv


_tc2 = pl.pallas_call(
    _tc2_body,
    grid=(G,),
    in_specs=[
        pl.BlockSpec((RB, 64), lambda i: (i, 0)),
        pl.BlockSpec((RB, 64), lambda i: (i, 0)),
        pl.BlockSpec((RB, 64), lambda i: (i, 0)),
        pl.BlockSpec((RB, 1), lambda i: (i, 0)),
        pl.BlockSpec((1, 64), lambda i: (0, 0)),
        pl.BlockSpec((64, 64), lambda i: (0, 0)),
    ],
    out_specs=pl.BlockSpec((RB, 64), lambda i: (i, 0)),
    out_shape=jax.ShapeDtypeStruct((N, 64), jnp.float32),
)


def _tc3_body(sa_r, sb_r, hp_r, dinv_r, b2_r, out_r):
    z = dinv_r[...] * (sa_r[...] + sb_r[...] + hp_r[...]) + b2_r[...]
    col = lax.broadcasted_iota(jnp.int32, (RB, 64), 1)
    zm = jnp.where(col < 40, z, -1e30)
    m = jnp.max(zm, axis=1, keepdims=True)
    e = jnp.exp(zm - m)
    s = jnp.sum(e, axis=1, keepdims=True)
    out_r[...] = zm - m - jnp.log(s)


_tc3 = pl.pallas_call(
    _tc3_body,
    grid=(G,),
    in_specs=[
        pl.BlockSpec((RB, 64), lambda i: (i, 0)),
        pl.BlockSpec((RB, 64), lambda i: (i, 0)),
        pl.BlockSpec((RB, 64), lambda i: (i, 0)),
        pl.BlockSpec((RB, 1), lambda i: (i, 0)),
        pl.BlockSpec((1, 64), lambda i: (0, 0)),
    ],
    out_specs=pl.BlockSpec((RB, 64), lambda i: (i, 0)),
    out_shape=jax.ShapeDtypeStruct((N, 64), jnp.float32),
)


@jax.jit
def kernel(x, edge_index, W1, b1, W2, b2):
    ei = edge_index.astype(jnp.int32)
    src = ei[0].reshape(NW, STEPS, K)
    dst = ei[1].reshape(NW, STEPS, K)

    cnt = _sc_count(dst)
    cnt = cnt.reshape(NC, NS * EPT_A)[:, :N]
    ca = cnt[0].reshape(N, 1)
    cb = cnt[1].reshape(N, 1)

    h1p, dinv = _tc1(x, W1, ca, cb)

    s1 = _sc_msg(h1p, src, dst).reshape(NC, N, 64)

    b1r = b1.reshape(1, 64)
    w2p = jnp.concatenate(
        [W2, jnp.zeros((64, 24), jnp.float32)], axis=1)
    b2p = jnp.concatenate([b2, jnp.zeros((24,), jnp.float32)]).reshape(1, 64)

    h2p = _tc2(s1[0], s1[1], h1p, dinv, b1r, w2p)

    s2 = _sc_msg(h2p, src, dst).reshape(NC, N, 64)

    out = _tc3(s2[0], s2[1], h2p, dinv, b2p)
    return out[:, :40]


# speedup vs baseline: 40.6396x; 1.0518x over previous
import functools

import jax
import jax.numpy as jnp
from jax import lax
from jax.experimental import pallas as pl
from jax.experimental.pallas import tpu as pltpu
from jax.experimental.pallas import tpu_sc as plsc

N = 10000
E = 320000
NC, NS, L = 2, 16, 16
NW = NC * NS
EPT = E // NW
K = 80
STEPS = EPT // K
NROWS_A = 640


_MESH = plsc.VectorSubcoreMesh(
    core_axis_name="c", subcore_axis_name="s",
    num_cores=NC, num_subcores=NS)
_SC_PARAMS = pltpu.CompilerParams(use_tc_tiling_on_sc=False)


def _zero_rows(buf, nrows_buf, ncols):
    zero = jnp.zeros((L,), jnp.float32)

    def zrow(r, carry):
        for c in range(ncols // L):
            buf[r, pl.ds(c * L, L)] = zero
        return carry

    lax.fori_loop(0, nrows_buf, zrow, 0)


ACC_A = NROWS_A * 16
EPT_A = ACC_A // NS
CNT_WIN = 8


def _count_body(dst_h, out_h, didx, ones_v, zbuf, acc, sem):
    cid = lax.axis_index("c")
    sid = lax.axis_index("s")
    wid = cid * NS + sid

    zero = jnp.zeros((L,), jnp.float32)
    one = jnp.ones((L,), jnp.float32)

    def fill(r, carry):
        zbuf[pl.ds(r * L, L)] = zero
        return carry

    lax.fori_loop(0, EPT_A // L, fill, 0)
    for i in range(K // L):
        ones_v[pl.ds(i * L, L)] = one
    pltpu.sync_copy(zbuf, acc.at[pl.ds(sid * EPT_A, EPT_A)])

    pltpu.sync_copy(dst_h.at[wid], didx)
    plsc.subcore_barrier()

    def s_start(j):
        pltpu.async_copy(ones_v, acc.at[didx.at[j]], sem, add=True)

    def s_wait(j):
        pltpu.make_async_copy(ones_v, acc.at[didx.at[j]], sem).wait()

    def step(j, carry):
        s_start(j)

        @pl.when(j >= CNT_WIN)
        def _():
            s_wait(j - CNT_WIN)

        return carry

    lax.fori_loop(0, STEPS, step, 0)
    for j in range(STEPS - CNT_WIN, STEPS):
        s_wait(j)

    plsc.subcore_barrier()
    pltpu.sync_copy(acc.at[pl.ds(sid * EPT_A, EPT_A)], out_h.at[wid])


_sc_count = pl.kernel(
    _count_body,
    out_type=jax.ShapeDtypeStruct((NW, EPT_A), jnp.float32),
    mesh=_MESH,
    compiler_params=_SC_PARAMS,
    scratch_types=[
        pltpu.VMEM((STEPS, K), jnp.int32),
        pltpu.VMEM((K,), jnp.float32),
        pltpu.VMEM((EPT_A,), jnp.float32),
        pltpu.VMEM_SHARED((ACC_A,), jnp.float32),
        pltpu.SemaphoreType.DMA,
    ],
)

D = 64
NBUF = 10
LAG = 5
ZR = 125


def _msg_body(table_h, src_h, dst_h, out_h, sidx, didx, msg, zbuf, acc,
              gsem, ssem):
    cid = lax.axis_index("c")
    sid = lax.axis_index("s")
    wid = cid * NS + sid
    rpt = N // NS

    _zero_rows(zbuf, ZR, D)
    for t in range(rpt // ZR):
        pltpu.sync_copy(zbuf, acc.at[pl.ds(sid * rpt + t * ZR, ZR)])

    pltpu.sync_copy(src_h.at[wid], sidx)
    pltpu.sync_copy(dst_h.at[wid], didx)
    plsc.subcore_barrier()

    def g_start(j, b):
        pltpu.async_copy(table_h.at[sidx.at[j]], msg.at[b], gsem.at[b])

    def g_wait(j, b):
        pltpu.make_async_copy(
            table_h.at[sidx.at[j]], msg.at[b], gsem.at[b]).wait()

    def s_start(j, b):
        pltpu.async_copy(msg.at[b], acc.at[didx.at[j]], ssem.at[b], add=True)

    def s_wait(j, b):
        pltpu.make_async_copy(
            msg.at[b], acc.at[didx.at[j]], ssem.at[b]).wait()

    for j in range(LAG):
        g_start(j, j % NBUF)
    for j in range(NBUF):
        b = j % NBUF
        bn = (b + LAG) % NBUF
        if j >= LAG:
            s_wait(j - LAG, bn)
        g_start(j + LAG, bn)
        g_wait(j, b)
        s_start(j, b)

    def group(i, carry):
        j0 = i * NBUF
        for b in range(NBUF):
            j = j0 + b
            bn = (b + LAG) % NBUF
            s_wait(j - LAG, bn)
            g_start(j + LAG, bn)
            g_wait(j, b)
            s_start(j, b)
        return carry

    lax.fori_loop(1, (STEPS - LAG) // NBUF, group, 0)
    for j in range(((STEPS - LAG) // NBUF) * NBUF, STEPS):
        b = j % NBUF
        s_wait(j - LAG, (b + LAG) % NBUF)
        if j + LAG < STEPS:
            g_start(j + LAG, (b + LAG) % NBUF)
        g_wait(j, b)
        s_start(j, b)
    for j in range(STEPS - LAG, STEPS):
        s_wait(j, j % NBUF)

    plsc.subcore_barrier()
    pltpu.sync_copy(acc.at[pl.ds(sid * rpt, rpt)], out_h.at[wid])


_sc_msg = pl.kernel(
    _msg_body,
    out_type=jax.ShapeDtypeStruct((NW, N // NS, D), jnp.float32),
    mesh=_MESH,
    compiler_params=_SC_PARAMS,
    scratch_types=[
        pltpu.VMEM((STEPS, K), jnp.int32),
        pltpu.VMEM((STEPS, K), jnp.int32),
        pltpu.VMEM((NBUF, K, D), jnp.float32),
        pltpu.VMEM((ZR, D), jnp.float32),
        pltpu.VMEM_SHARED((N, D), jnp.float32),
        pltpu.SemaphoreType.DMA((NBUF,)),
        pltpu.SemaphoreType.DMA((NBUF,)),
    ],
)

RB = 1000
G = N // RB


GT = NS
RT = N // NS


def _tc1_body(x_r, w_r, ca_r, cb_r, h_r, dinv_r):
    deg = ca_r[0] + cb_r[0] + 1.0
    dinv = lax.rsqrt(deg)
    h = jnp.dot(x_r[0], w_r[...], preferred_element_type=jnp.float32)
    dinv_r[0] = dinv
    h_r[0] = h * dinv


_tc1 = pl.pallas_call(
    _tc1_body,
    grid=(GT,),
    in_specs=[
        pl.BlockSpec((1, RT, 128), lambda i: (i, 0, 0)),
        pl.BlockSpec((128, 64), lambda i: (0, 0)),
        pl.BlockSpec((1, RT, 1), lambda i: (i, 0, 0)),
        pl.BlockSpec((1, RT, 1), lambda i: (i, 0, 0)),
    ],
    out_specs=[
        pl.BlockSpec((1, RT, 64), lambda i: (i, 0, 0)),
        pl.BlockSpec((1, RT, 1), lambda i: (i, 0, 0)),
    ],
    out_shape=[
        jax.ShapeDtypeStruct((GT, RT, 64), jnp.float32),
        jax.ShapeDtypeStruct((GT, RT, 1), jnp.float32),
    ],
)


def _tc2_body(sa_r, sb_r, hp_r, dinv_r, b1_r, w2_r, out_r):
    dinv = dinv_r[0]
    z = dinv * (sa_r[0] + sb_r[0] + hp_r[0]) + b1_r[...]
    z = jnp.maximum(z, 0.0)
    h2 = jnp.dot(z, w2_r[...], preferred_element_type=jnp.float32)
    out_r[0] = h2 * dinv


_tc2 = pl.pallas_call(
    _tc2_body,
    grid=(GT,),
    in_specs=[
        pl.BlockSpec((1, RT, 64), lambda i: (i, 0, 0)),
        pl.BlockSpec((1, RT, 64), lambda i: (i + GT, 0, 0)),
        pl.BlockSpec((1, RT, 64), lambda i: (i, 0, 0)),
        pl.BlockSpec((1, RT, 1), lambda i: (i, 0, 0)),
        pl.BlockSpec((1, 64), lambda i: (0, 0)),
        pl.BlockSpec((64, 64), lambda i: (0, 0)),
    ],
    out_specs=pl.BlockSpec((1, RT, 64), lambda i: (i, 0, 0)),
    out_shape=jax.ShapeDtypeStruct((GT, RT, 64), jnp.float32),
)


def _tc3_body(sa_r, sb_r, hp_r, dinv_r, b2_r, out_r):
    z = dinv_r[0] * (sa_r[0] + sb_r[0] + hp_r[0]) + b2_r[...]
    col = lax.broadcasted_iota(jnp.int32, (RT, 64), 1)
    zm = jnp.where(col < 40, z, -1e30)
    m = jnp.max(zm, axis=1, keepdims=True)
    e = jnp.exp(zm - m)
    s = jnp.sum(e, axis=1, keepdims=True)
    ls = zm - m - jnp.log(s)
    out_r[0] = ls[:, :40]


_tc3 = pl.pallas_call(
    _tc3_body,
    grid=(GT,),
    in_specs=[
        pl.BlockSpec((1, RT, 64), lambda i: (i, 0, 0)),
        pl.BlockSpec((1, RT, 64), lambda i: (i + GT, 0, 0)),
        pl.BlockSpec((1, RT, 64), lambda i: (i, 0, 0)),
        pl.BlockSpec((1, RT, 1), lambda i: (i, 0, 0)),
        pl.BlockSpec((1, 64), lambda i: (0, 0)),
    ],
    out_specs=pl.BlockSpec((1, RT, 40), lambda i: (i, 0, 0)),
    out_shape=jax.ShapeDtypeStruct((GT, RT, 40), jnp.float32),
)


@jax.jit
def kernel(x, edge_index, W1, b1, W2, b2):
    ei = edge_index.astype(jnp.int32)
    src = ei[0].reshape(NW, STEPS, K)
    dst = ei[1].reshape(NW, STEPS, K)

    cnt = _sc_count(dst)
    cnt = cnt.reshape(NC, NS * EPT_A)[:, :N]
    ca = cnt[0].reshape(GT, RT, 1)
    cb = cnt[1].reshape(GT, RT, 1)

    x16 = x.reshape(GT, RT, 128)
    h1p, dinv = _tc1(x16, W1, ca, cb)

    s1 = _sc_msg(h1p.reshape(N, D), src, dst)

    b1r = b1.reshape(1, 64)
    w2p = jnp.concatenate(
        [W2, jnp.zeros((64, 24), jnp.float32)], axis=1)
    b2p = jnp.concatenate([b2, jnp.zeros((24,), jnp.float32)]).reshape(1, 64)

    h2p = _tc2(s1, s1, h1p, dinv, b1r, w2p)

    s2 = _sc_msg(h2p.reshape(N, D), src, dst)

    out = _tc3(s2, s2, h2p, dinv, b2p)
    return out.reshape(N, 40)


# speedup vs baseline: 40.7747x; 1.0033x over previous
import functools

import jax
import jax.numpy as jnp
from jax import lax
from jax.experimental import pallas as pl
from jax.experimental.pallas import tpu as pltpu
from jax.experimental.pallas import tpu_sc as plsc

N = 10000
E = 320000
NC, NS, L = 2, 16, 16
NW = NC * NS
EPT = E // NW
K = 80
STEPS = EPT // K
NROWS_A = 640


_MESH = plsc.VectorSubcoreMesh(
    core_axis_name="c", subcore_axis_name="s",
    num_cores=NC, num_subcores=NS)
_SC_PARAMS = pltpu.CompilerParams(use_tc_tiling_on_sc=False)


def _zero_rows(buf, nrows_buf, ncols):
    zero = jnp.zeros((L,), jnp.float32)

    def zrow(r, carry):
        for c in range(ncols // L):
            buf[r, pl.ds(c * L, L)] = zero
        return carry

    lax.fori_loop(0, nrows_buf, zrow, 0)


ACC_A = NROWS_A * 16
EPT_A = ACC_A // NS
CNT_WIN = 8


def _count_body(dst_h, out_h, didx, ones_v, zbuf, acc, sem):
    cid = lax.axis_index("c")
    sid = lax.axis_index("s")
    wid = cid * NS + sid

    zero = jnp.zeros((L,), jnp.float32)
    one = jnp.ones((L,), jnp.float32)

    def fill(r, carry):
        zbuf[pl.ds(r * L, L)] = zero
        return carry

    lax.fori_loop(0, EPT_A // L, fill, 0)
    for i in range(K // L):
        ones_v[pl.ds(i * L, L)] = one
    pltpu.sync_copy(zbuf, acc.at[pl.ds(sid * EPT_A, EPT_A)])

    pltpu.sync_copy(dst_h.at[wid], didx)
    plsc.subcore_barrier()

    def s_start(j):
        pltpu.async_copy(ones_v, acc.at[didx.at[j]], sem, add=True)

    def s_wait(j):
        pltpu.make_async_copy(ones_v, acc.at[didx.at[j]], sem).wait()

    def step(j, carry):
        s_start(j)

        @pl.when(j >= CNT_WIN)
        def _():
            s_wait(j - CNT_WIN)

        return carry

    lax.fori_loop(0, STEPS, step, 0)
    for j in range(STEPS - CNT_WIN, STEPS):
        s_wait(j)

    plsc.subcore_barrier()
    pltpu.sync_copy(acc.at[pl.ds(sid * EPT_A, EPT_A)], out_h.at[wid])


_sc_count = pl.kernel(
    _count_body,
    out_type=jax.ShapeDtypeStruct((NW, EPT_A), jnp.float32),
    mesh=_MESH,
    compiler_params=_SC_PARAMS,
    scratch_types=[
        pltpu.VMEM((STEPS, K), jnp.int32),
        pltpu.VMEM((K,), jnp.float32),
        pltpu.VMEM((EPT_A,), jnp.float32),
        pltpu.VMEM_SHARED((ACC_A,), jnp.float32),
        pltpu.SemaphoreType.DMA,
    ],
)

D = 64
NBUF = 12
LAG = 6
ZR = 125


def _msg_body(table_h, src_h, dst_h, out_h, sidx, didx, msg, zbuf, acc,
              gsem, ssem):
    cid = lax.axis_index("c")
    sid = lax.axis_index("s")
    wid = cid * NS + sid
    rpt = N // NS

    _zero_rows(zbuf, ZR, D)
    for t in range(rpt // ZR):
        pltpu.sync_copy(zbuf, acc.at[pl.ds(sid * rpt + t * ZR, ZR)])

    pltpu.sync_copy(src_h.at[wid], sidx)
    pltpu.sync_copy(dst_h.at[wid], didx)
    plsc.subcore_barrier()

    def g_start(j, b):
        pltpu.async_copy(table_h.at[sidx.at[j]], msg.at[b], gsem.at[b])

    def g_wait(j, b):
        pltpu.make_async_copy(
            table_h.at[sidx.at[j]], msg.at[b], gsem.at[b]).wait()

    def s_start(j, b):
        pltpu.async_copy(msg.at[b], acc.at[didx.at[j]], ssem.at[b], add=True)

    def s_wait(j, b):
        pltpu.make_async_copy(
            msg.at[b], acc.at[didx.at[j]], ssem.at[b]).wait()

    for j in range(LAG):
        g_start(j, j % NBUF)
    for j in range(NBUF):
        b = j % NBUF
        bn = (b + LAG) % NBUF
        if j >= LAG:
            s_wait(j - LAG, bn)
        g_start(j + LAG, bn)
        g_wait(j, b)
        s_start(j, b)

    def group(i, carry):
        j0 = i * NBUF
        for b in range(NBUF):
            j = j0 + b
            bn = (b + LAG) % NBUF
            s_wait(j - LAG, bn)
            g_start(j + LAG, bn)
            g_wait(j, b)
            s_start(j, b)
        return carry

    lax.fori_loop(1, (STEPS - LAG) // NBUF, group, 0)
    for j in range(((STEPS - LAG) // NBUF) * NBUF, STEPS):
        b = j % NBUF
        s_wait(j - LAG, (b + LAG) % NBUF)
        if j + LAG < STEPS:
            g_start(j + LAG, (b + LAG) % NBUF)
        g_wait(j, b)
        s_start(j, b)
    for j in range(STEPS - LAG, STEPS):
        s_wait(j, j % NBUF)

    plsc.subcore_barrier()
    pltpu.sync_copy(acc.at[pl.ds(sid * rpt, rpt)], out_h.at[wid])


_sc_msg = pl.kernel(
    _msg_body,
    out_type=jax.ShapeDtypeStruct((NW, N // NS, D), jnp.float32),
    mesh=_MESH,
    compiler_params=_SC_PARAMS,
    scratch_types=[
        pltpu.VMEM((STEPS, K), jnp.int32),
        pltpu.VMEM((STEPS, K), jnp.int32),
        pltpu.VMEM((NBUF, K, D), jnp.float32),
        pltpu.VMEM((ZR, D), jnp.float32),
        pltpu.VMEM_SHARED((N, D), jnp.float32),
        pltpu.SemaphoreType.DMA((NBUF,)),
        pltpu.SemaphoreType.DMA((NBUF,)),
    ],
)

RB = 1000
G = N // RB


GT = NS
RT = N // NS


def _tc1_body(x_r, w_r, ca_r, cb_r, h_r, dinv_r):
    deg = ca_r[0] + cb_r[0] + 1.0
    dinv = lax.rsqrt(deg)
    h = jnp.dot(x_r[0], w_r[...], preferred_element_type=jnp.float32)
    dinv_r[0] = dinv
    h_r[0] = h * dinv


_tc1 = pl.pallas_call(
    _tc1_body,
    grid=(GT,),
    in_specs=[
        pl.BlockSpec((1, RT, 128), lambda i: (i, 0, 0)),
        pl.BlockSpec((128, 64), lambda i: (0, 0)),
        pl.BlockSpec((1, RT, 1), lambda i: (i, 0, 0)),
        pl.BlockSpec((1, RT, 1), lambda i: (i, 0, 0)),
    ],
    out_specs=[
        pl.BlockSpec((1, RT, 64), lambda i: (i, 0, 0)),
        pl.BlockSpec((1, RT, 1), lambda i: (i, 0, 0)),
    ],
    out_shape=[
        jax.ShapeDtypeStruct((GT, RT, 64), jnp.float32),
        jax.ShapeDtypeStruct((GT, RT, 1), jnp.float32),
    ],
)


def _tc2_body(sa_r, sb_r, hp_r, dinv_r, b1_r, w2_r, out_r):
    dinv = dinv_r[0]
    z = dinv * (sa_r[0] + sb_r[0] + hp_r[0]) + b1_r[...]
    z = jnp.maximum(z, 0.0)
    h2 = jnp.dot(z, w2_r[...], preferred_element_type=jnp.float32)
    out_r[0] = h2 * dinv


_tc2 = pl.pallas_call(
    _tc2_body,
    grid=(GT,),
    in_specs=[
        pl.BlockSpec((1, RT, 64), lambda i: (i, 0, 0)),
        pl.BlockSpec((1, RT, 64), lambda i: (i + GT, 0, 0)),
        pl.BlockSpec((1, RT, 64), lambda i: (i, 0, 0)),
        pl.BlockSpec((1, RT, 1), lambda i: (i, 0, 0)),
        pl.BlockSpec((1, 64), lambda i: (0, 0)),
        pl.BlockSpec((64, 64), lambda i: (0, 0)),
    ],
    out_specs=pl.BlockSpec((1, RT, 64), lambda i: (i, 0, 0)),
    out_shape=jax.ShapeDtypeStruct((GT, RT, 64), jnp.float32),
)


def _tc3_body(sa_r, sb_r, hp_r, dinv_r, b2_r, out_r):
    z = dinv_r[0] * (sa_r[0] + sb_r[0] + hp_r[0]) + b2_r[...]
    col = lax.broadcasted_iota(jnp.int32, (RT, 64), 1)
    zm = jnp.where(col < 40, z, -1e30)
    m = jnp.max(zm, axis=1, keepdims=True)
    e = jnp.exp(zm - m)
    s = jnp.sum(e, axis=1, keepdims=True)
    ls = zm - m - jnp.log(s)
    out_r[0] = ls[:, :40]


_tc3 = pl.pallas_call(
    _tc3_body,
    grid=(GT,),
    in_specs=[
        pl.BlockSpec((1, RT, 64), lambda i: (i, 0, 0)),
        pl.BlockSpec((1, RT, 64), lambda i: (i + GT, 0, 0)),
        pl.BlockSpec((1, RT, 64), lambda i: (i, 0, 0)),
        pl.BlockSpec((1, RT, 1), lambda i: (i, 0, 0)),
        pl.BlockSpec((1, 64), lambda i: (0, 0)),
    ],
    out_specs=pl.BlockSpec((1, RT, 40), lambda i: (i, 0, 0)),
    out_shape=jax.ShapeDtypeStruct((GT, RT, 40), jnp.float32),
)


@jax.jit
def kernel(x, edge_index, W1, b1, W2, b2):
    ei = edge_index.astype(jnp.int32)
    src = ei[0].reshape(NW, STEPS, K)
    dst = ei[1].reshape(NW, STEPS, K)

    cnt = _sc_count(dst)
    cnt = cnt.reshape(NC, NS * EPT_A)[:, :N]
    ca = cnt[0].reshape(GT, RT, 1)
    cb = cnt[1].reshape(GT, RT, 1)

    x16 = x.reshape(GT, RT, 128)
    h1p, dinv = _tc1(x16, W1, ca, cb)

    s1 = _sc_msg(h1p.reshape(N, D), src, dst)

    b1r = b1.reshape(1, 64)
    w2p = jnp.concatenate(
        [W2, jnp.zeros((64, 24), jnp.float32)], axis=1)
    b2p = jnp.concatenate([b2, jnp.zeros((24,), jnp.float32)]).reshape(1, 64)

    h2p = _tc2(s1, s1, h1p, dinv, b1r, w2p)

    s2 = _sc_msg(h2p.reshape(N, D), src, dst)

    out = _tc3(s2, s2, h2p, dinv, b2p)
    return out.reshape(N, 40)
